# Initial kernel scaffold; baseline (speedup 1.0000x reference)
import jax
import jax.numpy as jnp
from jax.experimental import pallas as pl


def kernel(x, router_w, w1, w2, w3):
    raise NotImplementedError("write your pallas kernel here")


# speedup vs baseline: 1.6820x; 1.6820x over previous
import functools

import jax
import jax.numpy as jnp
from jax import lax
from jax.experimental import pallas as pl
from jax.experimental.pallas import tpu as pltpu

E = 8
K = 2
T = 2048
D = 768
HID = 2048
LB_COEF = 0.01
Z_COEF = 0.001

TILE = 256
NT = (T * K) // TILE + (E - 1)
P = NT * TILE


def _router_plan_kernel(x_ref, rw_ref, dest_ref, gpair_ref, te_ref, aux_ref):
    x = x_ref[...]
    rw = rw_ref[...]
    logits = lax.dot_general(x, rw, (((1,), (1,)), ((), ())),
                             preferred_element_type=jnp.float32)

    e_ids = lax.broadcasted_iota(jnp.int32, (T, E), 1)
    m1 = jnp.max(logits, axis=1, keepdims=True)
    i1 = jnp.min(jnp.where(logits == m1, e_ids, E), axis=1, keepdims=True)
    masked = jnp.where(e_ids == i1, -jnp.inf, logits)
    m2 = jnp.max(masked, axis=1, keepdims=True)
    i2 = jnp.min(jnp.where(masked == m2, e_ids, E), axis=1, keepdims=True)

    g1 = 1.0 / (1.0 + jnp.exp(m2 - m1))
    g2 = 1.0 - g1

    one0 = (e_ids == i1).astype(jnp.float32)
    one1 = (e_ids == i2).astype(jnp.float32)

    ex = jnp.exp(logits - m1)
    sum_ex = jnp.sum(ex, axis=1, keepdims=True)
    probs = ex / sum_ex
    lse = m1 + jnp.log(sum_ex)
    z_loss = Z_COEF * jnp.mean(lse * lse, keepdims=True)
    counts = jnp.sum(one0 + one1, axis=0, keepdims=True)
    f = counts / float(T * K)
    p = jnp.mean(probs, axis=0, keepdims=True)
    lb_loss = LB_COEF * E * jnp.sum(f * p, keepdims=True)
    aux_ref[...] = lb_loss + z_loss

    m_all = one0 + one1
    r_iota = lax.broadcasted_iota(jnp.int32, (TILE, TILE), 0)
    c_iota = lax.broadcasted_iota(jnp.int32, (TILE, TILE), 1)
    w_tri = (c_iota < r_iota).astype(jnp.float32)
    s_blocks = []
    off = jnp.zeros((1, E), jnp.float32)
    for b in range(T // TILE):
        mb = m_all[b * TILE:(b + 1) * TILE, :]
        sb = lax.dot_general(w_tri, mb, (((1,), (0,)), ((), ())),
                             preferred_element_type=jnp.float32)
        s_blocks.append(sb + off)
        off = off + jnp.sum(mb, axis=0, keepdims=True)
    s_all = jnp.concatenate(s_blocks, axis=0)

    cnt_i = off.astype(jnp.int32)
    nt_e = (cnt_i + (TILE - 1)) >> 8
    u_tri = (lax.broadcasted_iota(jnp.int32, (E, E), 0)
             <= lax.broadcasted_iota(jnp.int32, (E, E), 1)).astype(jnp.float32)
    cum_end = lax.dot_general(nt_e.astype(jnp.float32), u_tri,
                              (((1,), (0,)), ((), ())),
                              preferred_element_type=jnp.float32)
    po = (cum_end - nt_e.astype(jnp.float32)) * float(TILE)

    rank0 = jnp.sum(s_all * one0, axis=1, keepdims=True)
    rank1 = jnp.sum(s_all * one1, axis=1, keepdims=True)
    base0 = jnp.sum(po * one0, axis=1, keepdims=True)
    base1 = jnp.sum(po * one1, axis=1, keepdims=True)
    dest0 = (base0 + rank0).astype(jnp.int32)
    dest1 = (base1 + rank1).astype(jnp.int32)
    dest_ref[...] = jnp.concatenate([dest0, dest1], axis=1)
    gpair_ref[...] = jnp.concatenate([g1, g2], axis=1)

    j_iota = lax.broadcasted_iota(jnp.int32, (1, 128), 1)
    acc = jnp.zeros((1, 128), jnp.int32)
    for e in range(E):
        ce = cum_end[0:1, e:e + 1].astype(jnp.int32)
        acc = acc + (j_iota >= ce).astype(jnp.int32)
    te_ref[...] = jnp.minimum(acc, E - 1)


def _router_plan(x_flat, router_w):
    return pl.pallas_call(
        _router_plan_kernel,
        out_shape=[
            jax.ShapeDtypeStruct((T, K), jnp.int32),
            jax.ShapeDtypeStruct((T, K), jnp.float32),
            jax.ShapeDtypeStruct((1, 128), jnp.int32),
            jax.ShapeDtypeStruct((1, 1), jnp.float32),
        ],
    )(x_flat, router_w)


def _ffn_kernel(te_ref, xs_ref, gs_ref, w1_ref, w3_ref, w2_ref, o_ref):
    del te_ref
    xs = xs_ref[...]
    a = lax.dot_general(xs, w1_ref[0], (((1,), (1,)), ((), ())),
                        preferred_element_type=jnp.float32)
    b = lax.dot_general(xs, w3_ref[0], (((1,), (1,)), ((), ())),
                        preferred_element_type=jnp.float32)
    c = (a * jax.nn.sigmoid(a)) * b
    c = c * gs_ref[...]
    o_ref[...] = lax.dot_general(c, w2_ref[0], (((1,), (1,)), ((), ())),
                                 preferred_element_type=jnp.float32)


def _grouped_ffn(xs, gate_sorted, tile_expert, w1, w2, w3):
    grid_spec = pltpu.PrefetchScalarGridSpec(
        num_scalar_prefetch=1,
        grid=(NT,),
        in_specs=[
            pl.BlockSpec((TILE, D), lambda j, te: (j, 0)),
            pl.BlockSpec((TILE, 1), lambda j, te: (j, 0)),
            pl.BlockSpec((1, HID, D), lambda j, te: (te[j], 0, 0)),
            pl.BlockSpec((1, HID, D), lambda j, te: (te[j], 0, 0)),
            pl.BlockSpec((1, D, HID), lambda j, te: (te[j], 0, 0)),
        ],
        out_specs=pl.BlockSpec((TILE, D), lambda j, te: (j, 0)),
    )
    return pl.pallas_call(
        _ffn_kernel,
        grid_spec=grid_spec,
        out_shape=jax.ShapeDtypeStruct((P, D), jnp.float32),
    )(tile_expert, xs, gate_sorted, w1, w3, w2)


def kernel(x, router_w, w1, w2, w3):
    x_flat = x.reshape(T, D)
    dest, gpair, te_pad, aux = _router_plan(x_flat, router_w)
    tile_expert = te_pad[0, :NT]

    dest_flat = dest.reshape(T * K)
    tok = (jnp.arange(T * K, dtype=jnp.int32) // K)
    src_token = jnp.full((P,), T, jnp.int32).at[dest_flat].set(tok)
    gate_sorted = jnp.zeros((P,), jnp.float32).at[dest_flat].set(
        gpair.reshape(T * K))

    x_pad = jnp.concatenate([x_flat, jnp.zeros((8, D), jnp.float32)], axis=0)
    xs = x_pad[src_token]

    out_s = _grouped_ffn(xs, gate_sorted.reshape(P, 1), tile_expert,
                         w1, w2, w3)

    y_flat = out_s[dest[:, 0]] + out_s[dest[:, 1]]
    return y_flat.reshape(1, T, D), aux.reshape(())
